# Initial kernel scaffold; baseline (speedup 1.0000x reference)
import functools

import jax
import jax.numpy as jnp
from jax import lax
from jax.experimental import pallas as pl
from jax.experimental.pallas import tpu as pltpu
from jax.experimental.pallas import tpu_sc as plsc

N = 100000
E = 1600000
W = 2000000
T = 6

C = 640
NCHUNK = W // C
NC = 2
NS = 16
NW = NC * NS
TRIPS = -(-NCHUNK // NW)
G = C // 16

_mesh = plsc.VectorSubcoreMesh(core_axis_name="c", subcore_axis_name="s")


@functools.partial(
    pl.kernel,
    out_type=jax.ShapeDtypeStruct((W, T), jnp.float32),
    mesh=_mesh,
    scratch_types=[
        pltpu.VMEM((N + 1,), jnp.int32),
        pltpu.VMEM((C, T), jnp.float32),
        pltpu.VMEM((C, T), jnp.float32),
        pltpu.VMEM((C,), jnp.int32),
        pltpu.VMEM((C,), jnp.int32),
        pltpu.VMEM((C,), jnp.int32),
        pltpu.VMEM((C,), jnp.int32),
        pltpu.SemaphoreType.DMA,
    ],
)
def _walk(indptr_hbm, indices_hbm, start_hbm, rand_hbm, out_hbm,
          indptr_v, rand_v, out_v, curr_v, addr_v, deg_v, nbr_v, sem):
    wid = lax.axis_index("s") * NC + lax.axis_index("c")
    pltpu.sync_copy(indptr_hbm, indptr_v)

    def trip(i, carry):
        c = wid + i * NW

        @pl.when(c < NCHUNK)
        def _():
            base = c * C
            pltpu.sync_copy(start_hbm.at[pl.ds(base, C)], curr_v)
            pltpu.sync_copy(rand_hbm.at[pl.ds(base, C)], rand_v)
            for t in range(T):
                def fwd(g, _, t=t):
                    sl = pl.ds(g * 16, 16)
                    lane = g * 16 + lax.iota(jnp.int32, 16)
                    tvec = jnp.full((16,), t, jnp.int32)
                    curr = curr_v[sl]
                    if t > 0:
                        curr = jnp.where(deg_v[sl] > 0, nbr_v[sl], curr)
                        curr_v[sl] = curr
                    plsc.store_scatter(out_v, [lane, tvec],
                                       curr.astype(jnp.float32))
                    if t < T - 1:
                        rs = plsc.load_gather(indptr_v, [curr])
                        re = plsc.load_gather(indptr_v, [curr + 1])
                        deg = re - rs
                        u = plsc.load_gather(rand_v, [lane, tvec])
                        off = (u * deg.astype(jnp.float32)).astype(jnp.int32)
                        off = jnp.minimum(off, jnp.maximum(deg - 1, 0))
                        off = jnp.maximum(off, 0)
                        addr_v[sl] = rs + off
                        deg_v[sl] = deg
                    return 0

                lax.fori_loop(0, G, fwd, 0)
                if t < T - 1:
                    pltpu.async_copy(indices_hbm.at[addr_v], nbr_v, sem).wait()
            pltpu.sync_copy(out_v, out_hbm.at[pl.ds(base, C)])

        return 0

    lax.fori_loop(0, TRIPS, trip, 0)


def kernel(indptr, indices, start_nodes, rand_u):
    return _walk(indptr.astype(jnp.int32), indices.astype(jnp.int32),
                 start_nodes.astype(jnp.int32), rand_u)


# speedup vs baseline: 40.0209x; 40.0209x over previous
import functools

import jax
import jax.numpy as jnp
from jax import lax
from jax.experimental import pallas as pl
from jax.experimental.pallas import tpu as pltpu
from jax.experimental.pallas import tpu_sc as plsc

N = 100000
E = 1600000
W = 2000000
T = 6

C = 640
NCHUNK = W // C
NC = 2
NS = 16
NW = NC * NS
TRIPS = -(-NCHUNK // NW)
G = C // 16

_mesh = plsc.VectorSubcoreMesh(core_axis_name="c", subcore_axis_name="s")


@functools.partial(
    pl.kernel,
    out_type=jax.ShapeDtypeStruct((W * T,), jnp.float32),
    mesh=_mesh,
    compiler_params=pltpu.CompilerParams(needs_layout_passes=False),
    scratch_types=[
        pltpu.VMEM((N + 1,), jnp.int32),
        pltpu.VMEM((C * T,), jnp.float32),
        pltpu.VMEM((C * T,), jnp.float32),
        pltpu.VMEM((C,), jnp.int32),
        pltpu.VMEM((C,), jnp.int32),
        pltpu.VMEM((C,), jnp.int32),
        pltpu.VMEM((C,), jnp.int32),
        pltpu.SemaphoreType.DMA,
    ],
)
def _walk(indptr_hbm, indices_hbm, start_hbm, rand_hbm, out_hbm,
          indptr_v, rand_v, out_v, curr_v, addr_v, deg_v, nbr_v, sem):
    wid = lax.axis_index("s") * NC + lax.axis_index("c")
    pltpu.sync_copy(indptr_hbm, indptr_v)

    def trip(i, carry):
        c = wid + i * NW

        @pl.when(c < NCHUNK)
        def _():
            base = c * C
            pltpu.sync_copy(start_hbm.at[pl.ds(base, C)], curr_v)
            pltpu.sync_copy(rand_hbm.at[pl.ds(base * T, C * T)], rand_v)
            for t in range(T):
                def fwd(g, _, t=t):
                    sl = pl.ds(g * 16, 16)
                    flat = (g * 16 + lax.iota(jnp.int32, 16)) * T + t
                    curr = curr_v[sl]
                    if t > 0:
                        curr = jnp.where(deg_v[sl] > 0, nbr_v[sl], curr)
                        curr_v[sl] = curr
                    plsc.store_scatter(out_v, [flat],
                                       curr.astype(jnp.float32))
                    if t < T - 1:
                        rs = plsc.load_gather(indptr_v, [curr])
                        re = plsc.load_gather(indptr_v, [curr + 1])
                        deg = re - rs
                        u = plsc.load_gather(rand_v, [flat])
                        off = (u * deg.astype(jnp.float32)).astype(jnp.int32)
                        off = jnp.minimum(off, jnp.maximum(deg - 1, 0))
                        off = jnp.maximum(off, 0)
                        addr_v[sl] = rs + off
                        deg_v[sl] = deg
                    return 0

                lax.fori_loop(0, G, fwd, 0)
                if t < T - 1:
                    pltpu.async_copy(indices_hbm.at[addr_v], nbr_v, sem).wait()
            pltpu.sync_copy(out_v, out_hbm.at[pl.ds(base * T, C * T)])

        return 0

    lax.fori_loop(0, TRIPS, trip, 0)


def kernel(indptr, indices, start_nodes, rand_u):
    flat = _walk(indptr.astype(jnp.int32), indices.astype(jnp.int32),
                 start_nodes.astype(jnp.int32), rand_u.reshape(W * T))
    return flat.reshape(W, T)


# speedup vs baseline: 47.7355x; 1.1928x over previous
import functools

import jax
import jax.numpy as jnp
from jax import lax
from jax.experimental import pallas as pl
from jax.experimental.pallas import tpu as pltpu
from jax.experimental.pallas import tpu_sc as plsc

N = 100000
E = 1600000
W = 2000000
T = 6

C = 640
NCHUNK = W // C
NC = 2
NS = 16
NW = NC * NS
TRIPS = -(-NCHUNK // (2 * NW))
G = C // 16
UNROLL = 4

_mesh = plsc.VectorSubcoreMesh(core_axis_name="c", subcore_axis_name="s")


@functools.partial(
    pl.kernel,
    out_type=jax.ShapeDtypeStruct((W * T,), jnp.float32),
    mesh=_mesh,
    compiler_params=pltpu.CompilerParams(needs_layout_passes=False),
    scratch_types=[
        pltpu.VMEM((N + 1,), jnp.int32),
        pltpu.VMEM((C * T,), jnp.float32),
        pltpu.VMEM((C * T,), jnp.float32),
        pltpu.VMEM((C,), jnp.int32),
        pltpu.VMEM((C,), jnp.int32),
        pltpu.VMEM((C,), jnp.int32),
        pltpu.VMEM((C,), jnp.int32),
        pltpu.VMEM((C * T,), jnp.float32),
        pltpu.VMEM((C * T,), jnp.float32),
        pltpu.VMEM((C,), jnp.int32),
        pltpu.VMEM((C,), jnp.int32),
        pltpu.VMEM((C,), jnp.int32),
        pltpu.VMEM((C,), jnp.int32),
        pltpu.SemaphoreType.DMA,
        pltpu.SemaphoreType.DMA,
    ],
)
def _walk(indptr_hbm, indices_hbm, start_hbm, rand_hbm, out_hbm,
          indptr_v,
          randA, outA, currA, addrA, degA, nbrA,
          randB, outB, currB, addrB, degB, nbrB,
          semA, semB):
    wid = lax.axis_index("s") * NC + lax.axis_index("c")
    pltpu.sync_copy(indptr_hbm, indptr_v)

    def compute(t, rand_v, out_v, curr_v, addr_v, deg_v, nbr_v):
        def body(g0, _):
            for k in range(UNROLL):
                g = g0 * UNROLL + k
                sl = pl.ds(g * 16, 16)
                flat = (g * 16 + lax.iota(jnp.int32, 16)) * T + t
                curr = curr_v[sl]
                if t > 0:
                    curr = jnp.where(deg_v[sl] > 0, nbr_v[sl], curr)
                    curr_v[sl] = curr
                plsc.store_scatter(out_v, [flat], curr.astype(jnp.float32))
                if t < T - 1:
                    rs = plsc.load_gather(indptr_v, [curr])
                    re = plsc.load_gather(indptr_v, [curr + 1])
                    deg = re - rs
                    u = plsc.load_gather(rand_v, [flat])
                    off = (u * deg.astype(jnp.float32)).astype(jnp.int32)
                    off = jnp.minimum(off, jnp.maximum(deg - 1, 0))
                    addr_v[sl] = rs + off
                    deg_v[sl] = deg
            return 0

        lax.fori_loop(0, G // UNROLL, body, 0)

    def trip(i, _):
        cA = jnp.minimum(wid + i * (2 * NW), NCHUNK - 1)
        cB = jnp.minimum(wid + NW + i * (2 * NW), NCHUNK - 1)
        baseA = cA * C
        baseB = cB * C
        inA0 = pltpu.async_copy(start_hbm.at[pl.ds(baseA, C)], currA, semA)
        inA1 = pltpu.async_copy(rand_hbm.at[pl.ds(baseA * T, C * T)], randA,
                                semA)
        inB0 = pltpu.async_copy(start_hbm.at[pl.ds(baseB, C)], currB, semB)
        inB1 = pltpu.async_copy(rand_hbm.at[pl.ds(baseB * T, C * T)], randB,
                                semB)
        inA0.wait()
        inA1.wait()
        gA = gB = None
        for t in range(T):
            if t > 0:
                gA.wait()
            compute(t, randA, outA, currA, addrA, degA, nbrA)
            if t < T - 1:
                gA = pltpu.async_copy(indices_hbm.at[addrA], nbrA, semA)
            if t == 0:
                inB0.wait()
                inB1.wait()
            else:
                gB.wait()
            compute(t, randB, outB, currB, addrB, degB, nbrB)
            if t < T - 1:
                gB = pltpu.async_copy(indices_hbm.at[addrB], nbrB, semB)
        oA = pltpu.async_copy(outA, out_hbm.at[pl.ds(baseA * T, C * T)], semA)
        oB = pltpu.async_copy(outB, out_hbm.at[pl.ds(baseB * T, C * T)], semB)
        oA.wait()
        oB.wait()
        return 0

    lax.fori_loop(0, TRIPS, trip, 0)


def kernel(indptr, indices, start_nodes, rand_u):
    flat = _walk(indptr.astype(jnp.int32), indices.astype(jnp.int32),
                 start_nodes.astype(jnp.int32), rand_u.reshape(W * T))
    return flat.reshape(W, T)


# speedup vs baseline: 49.9370x; 1.0461x over previous
import functools

import jax
import jax.numpy as jnp
from jax import lax
from jax.experimental import pallas as pl
from jax.experimental.pallas import tpu as pltpu
from jax.experimental.pallas import tpu_sc as plsc

N = 100000
E = 1600000
W = 2000000
T = 6

C = 640
NCHUNK = W // C
NC = 2
NS = 16
NW = NC * NS
TRIPS = -(-NCHUNK // (2 * NW))
G = C // 16
UNROLL = 8

_mesh = plsc.VectorSubcoreMesh(core_axis_name="c", subcore_axis_name="s")


@functools.partial(
    pl.kernel,
    out_type=jax.ShapeDtypeStruct((W * T,), jnp.float32),
    mesh=_mesh,
    compiler_params=pltpu.CompilerParams(needs_layout_passes=False),
    scratch_types=[
        pltpu.VMEM((N + 1,), jnp.int32),
        pltpu.VMEM((C * T,), jnp.float32),
        pltpu.VMEM((C * T,), jnp.float32),
        pltpu.VMEM((C,), jnp.int32),
        pltpu.VMEM((C,), jnp.int32),
        pltpu.VMEM((C,), jnp.int32),
        pltpu.VMEM((C,), jnp.int32),
        pltpu.VMEM((C * T,), jnp.float32),
        pltpu.VMEM((C * T,), jnp.float32),
        pltpu.VMEM((C,), jnp.int32),
        pltpu.VMEM((C,), jnp.int32),
        pltpu.VMEM((C,), jnp.int32),
        pltpu.VMEM((C,), jnp.int32),
        pltpu.SemaphoreType.DMA,
        pltpu.SemaphoreType.DMA,
    ],
)
def _walk(indptr_hbm, indices_hbm, start_hbm, rand_hbm, out_hbm,
          indptr_v,
          randA, outA, currA, addrA, degA, nbrA,
          randB, outB, currB, addrB, degB, nbrB,
          semA, semB):
    wid = lax.axis_index("s") * NC + lax.axis_index("c")
    pltpu.sync_copy(indptr_hbm, indptr_v)

    def compute(t, rand_v, out_v, curr_v, addr_v, deg_v, nbr_v):
        @plsc.parallel_loop(0, G, 1, unroll=UNROLL)
        def body(g):
            sl = pl.ds(g * 16, 16)
            flat = (g * 16 + lax.iota(jnp.int32, 16)) * T + t
            curr = curr_v[sl]
            if t > 0:
                curr = jnp.where(deg_v[sl] > 0, nbr_v[sl], curr)
                curr_v[sl] = curr
            plsc.store_scatter(out_v, [flat], curr.astype(jnp.float32))
            if t < T - 1:
                rs = plsc.load_gather(indptr_v, [curr])
                re = plsc.load_gather(indptr_v, [curr + 1])
                deg = re - rs
                u = plsc.load_gather(rand_v, [flat])
                off = (u * deg.astype(jnp.float32)).astype(jnp.int32)
                off = jnp.minimum(off, jnp.maximum(deg - 1, 0))
                addr_v[sl] = rs + off
                deg_v[sl] = deg

    def trip(i, _):
        cA = jnp.minimum(wid + i * (2 * NW), NCHUNK - 1)
        cB = jnp.minimum(wid + NW + i * (2 * NW), NCHUNK - 1)
        baseA = cA * C
        baseB = cB * C
        inA0 = pltpu.async_copy(start_hbm.at[pl.ds(baseA, C)], currA, semA)
        inA1 = pltpu.async_copy(rand_hbm.at[pl.ds(baseA * T, C * T)], randA,
                                semA)
        inB0 = pltpu.async_copy(start_hbm.at[pl.ds(baseB, C)], currB, semB)
        inB1 = pltpu.async_copy(rand_hbm.at[pl.ds(baseB * T, C * T)], randB,
                                semB)
        inA0.wait()
        inA1.wait()
        gA = gB = None
        for t in range(T):
            if t > 0:
                gA.wait()
            compute(t, randA, outA, currA, addrA, degA, nbrA)
            if t < T - 1:
                gA = pltpu.async_copy(indices_hbm.at[addrA], nbrA, semA)
            if t == 0:
                inB0.wait()
                inB1.wait()
            else:
                gB.wait()
            compute(t, randB, outB, currB, addrB, degB, nbrB)
            if t < T - 1:
                gB = pltpu.async_copy(indices_hbm.at[addrB], nbrB, semB)
        oA = pltpu.async_copy(outA, out_hbm.at[pl.ds(baseA * T, C * T)], semA)
        oB = pltpu.async_copy(outB, out_hbm.at[pl.ds(baseB * T, C * T)], semB)
        oA.wait()
        oB.wait()
        return 0

    lax.fori_loop(0, TRIPS, trip, 0)


def kernel(indptr, indices, start_nodes, rand_u):
    flat = _walk(indptr.astype(jnp.int32), indices.astype(jnp.int32),
                 start_nodes.astype(jnp.int32), rand_u.reshape(W * T))
    return flat.reshape(W, T)


# speedup vs baseline: 218.0439x; 4.3664x over previous
import functools

import jax
import jax.numpy as jnp
from jax import lax
from jax.experimental import pallas as pl
from jax.experimental.pallas import tpu as pltpu
from jax.experimental.pallas import tpu_sc as plsc

N = 100000
E = 1600000
W = 2000000
T = 6

C = 640
NCHUNK = W // C
NC = 2
NS = 16
NW = NC * NS
TRIPS = -(-NCHUNK // (2 * NW))
G = C // 16
UNROLL = 8

_mesh = plsc.VectorSubcoreMesh(core_axis_name="c", subcore_axis_name="s")


@functools.partial(
    pl.kernel,
    out_type=jax.ShapeDtypeStruct((W * T,), jnp.float32),
    mesh=_mesh,
    compiler_params=pltpu.CompilerParams(needs_layout_passes=False),
    scratch_types=[
        pltpu.VMEM((N + 1,), jnp.int32),
        pltpu.VMEM((C * T,), jnp.float32),
        pltpu.VMEM((C * T,), jnp.float32),
        pltpu.VMEM((C,), jnp.int32),
        pltpu.VMEM((C,), jnp.int32),
        pltpu.VMEM((C,), jnp.int32),
        pltpu.VMEM((C,), jnp.int32),
        pltpu.VMEM((C * T,), jnp.float32),
        pltpu.VMEM((C * T,), jnp.float32),
        pltpu.VMEM((C,), jnp.int32),
        pltpu.VMEM((C,), jnp.int32),
        pltpu.VMEM((C,), jnp.int32),
        pltpu.VMEM((C,), jnp.int32),
        pltpu.SemaphoreType.DMA,
        pltpu.SemaphoreType.DMA,
    ],
)
def _walk(indptr_hbm, indices_hbm, start_hbm, rand_hbm, out_hbm,
          indptr_v,
          randA, outA, currA, addrA, degA, nbrA,
          randB, outB, currB, addrB, degB, nbrB,
          semA, semB):
    wid = lax.axis_index("s") * NC + lax.axis_index("c")
    pltpu.sync_copy(indptr_hbm, indptr_v)

    def compute(t, rand_v, out_v, curr_v, addr_v, deg_v, nbr_v):
        @plsc.parallel_loop(0, G, 1, unroll=UNROLL)
        def body(g):
            sl = pl.ds(g * 16, 16)
            tsl = pl.ds(t * C + g * 16, 16)
            curr = curr_v[sl]
            if t > 0:
                curr = jnp.where(deg_v[sl] > 0, nbr_v[sl], curr)
                curr_v[sl] = curr
            out_v[tsl] = curr.astype(jnp.float32)
            if t < T - 1:
                rs = plsc.load_gather(indptr_v, [curr])
                re = plsc.load_gather(indptr_v, [curr + 1])
                deg = re - rs
                u = rand_v[tsl]
                off = (u * deg.astype(jnp.float32)).astype(jnp.int32)
                off = jnp.minimum(off, jnp.maximum(deg - 1, 0))
                addr_v[sl] = rs + off
                deg_v[sl] = deg

    def trip(i, _):
        cA = jnp.minimum(wid + i * (2 * NW), NCHUNK - 1)
        cB = jnp.minimum(wid + NW + i * (2 * NW), NCHUNK - 1)
        baseA = cA * C
        baseB = cB * C
        inA0 = pltpu.async_copy(start_hbm.at[pl.ds(baseA, C)], currA, semA)
        inA1 = pltpu.async_copy(rand_hbm.at[pl.ds(baseA * T, C * T)], randA,
                                semA)
        inB0 = pltpu.async_copy(start_hbm.at[pl.ds(baseB, C)], currB, semB)
        inB1 = pltpu.async_copy(rand_hbm.at[pl.ds(baseB * T, C * T)], randB,
                                semB)
        inA0.wait()
        inA1.wait()
        gA = gB = None
        for t in range(T):
            if t > 0:
                gA.wait()
            compute(t, randA, outA, currA, addrA, degA, nbrA)
            if t < T - 1:
                gA = pltpu.async_copy(indices_hbm.at[addrA], nbrA, semA)
            if t == 0:
                inB0.wait()
                inB1.wait()
            else:
                gB.wait()
            compute(t, randB, outB, currB, addrB, degB, nbrB)
            if t < T - 1:
                gB = pltpu.async_copy(indices_hbm.at[addrB], nbrB, semB)
        oA = pltpu.async_copy(outA, out_hbm.at[pl.ds(baseA * T, C * T)], semA)
        oB = pltpu.async_copy(outB, out_hbm.at[pl.ds(baseB * T, C * T)], semB)
        oA.wait()
        oB.wait()
        return 0

    lax.fori_loop(0, TRIPS, trip, 0)


def kernel(indptr, indices, start_nodes, rand_u):
    rand_t = rand_u.reshape(NCHUNK, C, T).transpose(0, 2, 1).reshape(W * T)
    flat = _walk(indptr.astype(jnp.int32), indices.astype(jnp.int32),
                 start_nodes.astype(jnp.int32), rand_t)
    return flat.reshape(NCHUNK, T, C).transpose(0, 2, 1).reshape(W, T)
